# baseline (speedup 1.0000x reference)
import functools
import math

import jax
import jax.numpy as jnp
import numpy as np
from jax import lax
from jax.experimental import pallas as pl
from jax.experimental.pallas import tpu as pltpu
from jax.experimental.pallas import tpu_sc as plsc

VOCAB = 100000
HIDDEN = 128
BASE = 10000.0
EPS = 1e-8
B, S = 4, 8192
N = B * S

MEAN_FREQ = float(np.mean(np.reciprocal(np.arange(1, VOCAB + 1, dtype=np.float32))))

NC, NS, L = 2, 16, 16
NW = NC * NS

CH = 128
B_PER_W = N // NW
N_CHUNKS = B_PER_W // CH

TBLK = 1024

PI2_HI = 6.28125
PI2_MID = float(np.float32(2 * np.pi - 6.28125))
PI2_LO = float(2 * np.pi - 6.28125 - np.float64(np.float32(2 * np.pi - 6.28125)))
INV2PI = float(1.0 / (2 * np.pi))
MAGIC = 12582912.0


def _fit_even(f, deg_pairs):
    x = np.linspace(-np.pi, np.pi, 40001)
    y = f(x)
    a = np.stack([x ** (2 * i) for i in range(deg_pairs)], axis=1)
    coef, *_ = np.linalg.lstsq(a, y, rcond=None)
    return [float(c) for c in coef]


_NC = 5
_COS_C = _fit_even(np.cos, _NC)
_SIN_C = _fit_even(lambda x: np.where(x == 0, 1.0, np.sin(x) / np.where(x == 0, 1.0, x)), _NC)


def _sincos(phase):
    u = phase * INV2PI
    k = lax.round(u, lax.RoundingMethod.TO_NEAREST_EVEN)
    r = phase - k * PI2_HI
    r = r - k * PI2_MID
    r2 = r * r
    c = jnp.full_like(r2, _COS_C[_NC - 1])
    s = jnp.full_like(r2, _SIN_C[_NC - 1])
    for i in range(_NC - 2, -1, -1):
        c = c * r2 + _COS_C[i]
        s = s * r2 + _SIN_C[i]
    return s * r, c


def _sc_gather_body(ids_hbm, table_hbm, ls_hbm, rows_out, scale_out,
                    idx0, idx1, rows0, rows1, ls0, ls1, scl0, scl1,
                    g0, g1, l0, l1, wr0, wr1, ws0, ws1):
    wid = lax.axis_index("s") * NC + lax.axis_index("c")
    base = wid * B_PER_W
    idx = (idx0, idx1)
    rows = (rows0, rows1)
    lsv = (ls0, ls1)
    scl = (scl0, scl1)
    sg = (g0, g1)
    sl = (l0, l1)
    swr = (wr0, wr1)
    sws = (ws0, ws1)

    def issue(i):
        b = i % 2
        off = base + i * CH
        pltpu.sync_copy(ids_hbm.at[pl.ds(off, CH)], idx[b])
        cg = pltpu.async_copy(table_hbm.at[idx[b]], rows[b], sg[b])
        cl = pltpu.async_copy(ls_hbm.at[idx[b]], lsv[b], sl[b])
        return cg, cl

    pend = {0: issue(0)}
    writes = {}
    for i in range(N_CHUNKS):
        b = i % 2
        if i + 1 < N_CHUNKS:
            if i >= 1:
                for w in writes.pop(i - 1):
                    w.wait()
            pend[i + 1] = issue(i + 1)
        cg, cl = pend.pop(i)
        cl.wait()
        for j in range(CH // L):
            ids16 = idx[b][pl.ds(j * L, L)]
            ls16 = lsv[b][pl.ds(j * L, L)]
            scl[b][pl.ds(j * L, L)] = (
                (ids16.astype(jnp.float32) + 1.0) * MEAN_FREQ * jnp.exp(ls16))
        cg.wait()
        off = base + i * CH
        wrow = pltpu.async_copy(rows[b], rows_out.at[pl.ds(off, CH)], swr[b])
        wscl = pltpu.async_copy(scl[b], scale_out.at[pl.ds(off, CH)], sws[b])
        writes[i] = (wrow, wscl)
    for i in list(writes):
        for w in writes.pop(i):
            w.wait()


@functools.lru_cache(maxsize=1)
def _sc_gather():
    return pl.kernel(
        _sc_gather_body,
        mesh=plsc.VectorSubcoreMesh(core_axis_name="c", subcore_axis_name="s"),
        out_type=[
            jax.ShapeDtypeStruct((N, HIDDEN), jnp.float32),
            jax.ShapeDtypeStruct((N,), jnp.float32),
        ],
        scratch_types=[
            pltpu.VMEM((CH,), jnp.int32),
            pltpu.VMEM((CH,), jnp.int32),
            pltpu.VMEM((CH, HIDDEN), jnp.float32),
            pltpu.VMEM((CH, HIDDEN), jnp.float32),
            pltpu.VMEM((CH,), jnp.float32),
            pltpu.VMEM((CH,), jnp.float32),
            pltpu.VMEM((CH,), jnp.float32),
            pltpu.VMEM((CH,), jnp.float32),
            pltpu.SemaphoreType.DMA,
            pltpu.SemaphoreType.DMA,
            pltpu.SemaphoreType.DMA,
            pltpu.SemaphoreType.DMA,
            pltpu.SemaphoreType.DMA,
            pltpu.SemaphoreType.DMA,
            pltpu.SemaphoreType.DMA,
            pltpu.SemaphoreType.DMA,
        ],
    )


def _tc_body(rows_ref, scale_ref, wt_ref, omega_ref, z_ref):
    i = pl.program_id(0)
    emb = rows_ref[...]
    LOG2E = 1.4426950408889634
    LN2 = 0.6931471805599453
    tp = jnp.maximum(emb, 0.0) + LN2 * jnp.log2(1.0 + jnp.exp2(-LOG2E * jnp.abs(emb)))
    nrm2 = jnp.dot(tp * tp, jnp.ones((HIDDEN, HIDDEN), jnp.float32),
                   preferred_element_type=jnp.float32)
    amp = tp * (scale_ref[...] * lax.rsqrt(jnp.maximum(nrm2, EPS)))
    pos0 = (i % (S // TBLK)) * TBLK
    pos = (pos0 + lax.broadcasted_iota(jnp.int32, (TBLK, 1), 0)).astype(jnp.float32)
    phase = pos * omega_ref[...] + jnp.dot(
        emb, wt_ref[...], preferred_element_type=jnp.float32)
    sn, cs = _sincos(phase)
    z_ref[0] = amp * cs
    z_ref[1] = amp * sn


def _tc_compute(rows, scale2, wt, omega):
    grid = (N // TBLK,)
    return pl.pallas_call(
        _tc_body,
        grid=grid,
        in_specs=[
            pl.BlockSpec((TBLK, HIDDEN), lambda i: (i, 0)),
            pl.BlockSpec((TBLK, 1), lambda i: (i, 0)),
            pl.BlockSpec((HIDDEN, HIDDEN), lambda i: (0, 0)),
            pl.BlockSpec((1, HIDDEN), lambda i: (0, 0)),
        ],
        out_specs=pl.BlockSpec((2, TBLK, HIDDEN), lambda i: (0, i, 0)),
        out_shape=jax.ShapeDtypeStruct((2, N, HIDDEN), jnp.float32),
    )(rows, scale2, wt, omega)


def kernel(token_ids, emb_table, log_scale_res, phase_proj_W):
    ids_flat = token_ids.reshape(N).astype(jnp.int32)
    ls_flat = log_scale_res.reshape(VOCAB)
    rows, scale = _sc_gather()(ids_flat, emb_table, ls_flat)
    scale2 = scale.reshape(N, 1)
    wt = phase_proj_W.T
    omega = (BASE ** (-jnp.arange(HIDDEN, dtype=jnp.float32) / HIDDEN)).reshape(1, HIDDEN)
    z = _tc_compute(rows, scale2, wt, omega)
    return lax.complex(z[0], z[1]).reshape(B, S, HIDDEN)


# speedup vs baseline: 1.0905x; 1.0905x over previous
import functools
import math

import jax
import jax.numpy as jnp
import numpy as np
from jax import lax
from jax.experimental import pallas as pl
from jax.experimental.pallas import tpu as pltpu
from jax.experimental.pallas import tpu_sc as plsc

VOCAB = 100000
HIDDEN = 128
BASE = 10000.0
EPS = 1e-8
B, S = 4, 8192
N = B * S

MEAN_FREQ = float(np.mean(np.reciprocal(np.arange(1, VOCAB + 1, dtype=np.float32))))

NC, NS, L = 2, 16, 16
NW = NC * NS

CH = 128
B_PER_W = N // NW
N_CHUNKS = B_PER_W // CH

TBLK = 2048

PI2_HI = 6.28125
PI2_MID = float(np.float32(2 * np.pi - 6.28125))
PI2_LO = float(2 * np.pi - 6.28125 - np.float64(np.float32(2 * np.pi - 6.28125)))
INV2PI = float(1.0 / (2 * np.pi))
MAGIC = 12582912.0


def _fit_even(f, deg_pairs):
    x = np.linspace(-np.pi, np.pi, 40001)
    y = f(x)
    a = np.stack([x ** (2 * i) for i in range(deg_pairs)], axis=1)
    coef, *_ = np.linalg.lstsq(a, y, rcond=None)
    return [float(c) for c in coef]


_NC = 4
_COS_C = _fit_even(np.cos, _NC)
_SIN_C = _fit_even(lambda x: np.where(x == 0, 1.0, np.sin(x) / np.where(x == 0, 1.0, x)), _NC)


def _sincos(phase):
    u = phase * INV2PI
    k = lax.round(u, lax.RoundingMethod.TO_NEAREST_EVEN)
    r = phase - k * PI2_HI
    r = r - k * PI2_MID
    r2 = r * r
    c = jnp.full_like(r2, _COS_C[_NC - 1])
    s = jnp.full_like(r2, _SIN_C[_NC - 1])
    for i in range(_NC - 2, -1, -1):
        c = c * r2 + _COS_C[i]
        s = s * r2 + _SIN_C[i]
    return s * r, c


def _sc_gather_body(ids_hbm, table_hbm, ls_hbm, rows_out, scale_out,
                    idx0, idx1, rows0, rows1, ls0, ls1, scl0, scl1,
                    g0, g1, l0, l1, wr0, wr1, ws0, ws1):
    wid = lax.axis_index("s") * NC + lax.axis_index("c")
    base = wid * B_PER_W
    idx = (idx0, idx1)
    rows = (rows0, rows1)
    lsv = (ls0, ls1)
    scl = (scl0, scl1)
    sg = (g0, g1)
    sl = (l0, l1)
    swr = (wr0, wr1)
    sws = (ws0, ws1)

    def issue(i):
        b = i % 2
        off = base + i * CH
        pltpu.sync_copy(ids_hbm.at[pl.ds(off, CH)], idx[b])
        cg = pltpu.async_copy(table_hbm.at[idx[b]], rows[b], sg[b])
        cl = pltpu.async_copy(ls_hbm.at[idx[b]], lsv[b], sl[b])
        return cg, cl

    pend = {0: issue(0)}
    writes = {}
    for i in range(N_CHUNKS):
        b = i % 2
        if i + 1 < N_CHUNKS:
            if i >= 1:
                for w in writes.pop(i - 1):
                    w.wait()
            pend[i + 1] = issue(i + 1)
        cg, cl = pend.pop(i)
        cl.wait()
        for j in range(CH // L):
            ids16 = idx[b][pl.ds(j * L, L)]
            ls16 = lsv[b][pl.ds(j * L, L)]
            scl[b][pl.ds(j * L, L)] = (
                (ids16.astype(jnp.float32) + 1.0) * MEAN_FREQ * jnp.exp(ls16))
        cg.wait()
        off = base + i * CH
        wrow = pltpu.async_copy(rows[b], rows_out.at[pl.ds(off, CH)], swr[b])
        wscl = pltpu.async_copy(scl[b], scale_out.at[pl.ds(off, CH)], sws[b])
        writes[i] = (wrow, wscl)
    for i in list(writes):
        for w in writes.pop(i):
            w.wait()


@functools.lru_cache(maxsize=1)
def _sc_gather():
    return pl.kernel(
        _sc_gather_body,
        mesh=plsc.VectorSubcoreMesh(core_axis_name="c", subcore_axis_name="s"),
        out_type=[
            jax.ShapeDtypeStruct((N, HIDDEN), jnp.float32),
            jax.ShapeDtypeStruct((N,), jnp.float32),
        ],
        scratch_types=[
            pltpu.VMEM((CH,), jnp.int32),
            pltpu.VMEM((CH,), jnp.int32),
            pltpu.VMEM((CH, HIDDEN), jnp.float32),
            pltpu.VMEM((CH, HIDDEN), jnp.float32),
            pltpu.VMEM((CH,), jnp.float32),
            pltpu.VMEM((CH,), jnp.float32),
            pltpu.VMEM((CH,), jnp.float32),
            pltpu.VMEM((CH,), jnp.float32),
            pltpu.SemaphoreType.DMA,
            pltpu.SemaphoreType.DMA,
            pltpu.SemaphoreType.DMA,
            pltpu.SemaphoreType.DMA,
            pltpu.SemaphoreType.DMA,
            pltpu.SemaphoreType.DMA,
            pltpu.SemaphoreType.DMA,
            pltpu.SemaphoreType.DMA,
        ],
    )


def _tc_body(rows_ref, scale_ref, wt_ref, omega_ref, re_ref, im_ref):
    i = pl.program_id(0)
    emb = rows_ref[...]
    LOG2E = 1.4426950408889634
    LN2 = 0.6931471805599453
    tp = jnp.maximum(emb, 0.0) + LN2 * jnp.log2(1.0 + jnp.exp2(-LOG2E * jnp.abs(emb)))
    nrm2 = jnp.dot(tp * tp, jnp.ones((HIDDEN, HIDDEN), jnp.float32),
                   preferred_element_type=jnp.float32)
    amp = tp * (scale_ref[...] * lax.rsqrt(jnp.maximum(nrm2, EPS)))
    pos0 = (i % (S // TBLK)) * TBLK
    pos = (pos0 + lax.broadcasted_iota(jnp.int32, (TBLK, 1), 0)).astype(jnp.float32)
    phase = pos * omega_ref[...] + jnp.dot(
        emb, wt_ref[...], preferred_element_type=jnp.float32)
    sn, cs = _sincos(phase)
    re_ref[...] = amp * cs
    im_ref[...] = amp * sn


def _tc_compute(rows, scale2, wt, omega):
    grid = (N // TBLK,)
    return pl.pallas_call(
        _tc_body,
        grid=grid,
        in_specs=[
            pl.BlockSpec((TBLK, HIDDEN), lambda i: (i, 0)),
            pl.BlockSpec((TBLK, 1), lambda i: (i, 0)),
            pl.BlockSpec((HIDDEN, HIDDEN), lambda i: (0, 0)),
            pl.BlockSpec((1, HIDDEN), lambda i: (0, 0)),
        ],
        out_specs=[
            pl.BlockSpec((TBLK, HIDDEN), lambda i: (i, 0)),
            pl.BlockSpec((TBLK, HIDDEN), lambda i: (i, 0)),
        ],
        out_shape=[
            jax.ShapeDtypeStruct((N, HIDDEN), jnp.float32),
            jax.ShapeDtypeStruct((N, HIDDEN), jnp.float32),
        ],
    )(rows, scale2, wt, omega)


def kernel(token_ids, emb_table, log_scale_res, phase_proj_W):
    ids_flat = token_ids.reshape(N).astype(jnp.int32)
    ls_flat = log_scale_res.reshape(VOCAB)
    rows, scale = _sc_gather()(ids_flat, emb_table, ls_flat)
    scale2 = scale.reshape(N, 1)
    wt = phase_proj_W.T
    omega = (BASE ** (-jnp.arange(HIDDEN, dtype=jnp.float32) / HIDDEN)).reshape(1, HIDDEN)
    re, im = _tc_compute(rows, scale2, wt, omega)
    return lax.complex(re, im).reshape(B, S, HIDDEN)
